# baseline (speedup 1.0000x reference)
import jax
import jax.numpy as jnp
from jax import lax
from jax.experimental import pallas as pl
from jax.experimental.pallas import tpu as pltpu
from jax.experimental.pallas import tpu_sc as plsc

NC = 2
NS = 16
L = 16
NW = NC * NS

BATCH = 16384
F = 32
B_PER_W = BATCH // NW
CHUNK = 128
N_CHUNKS = B_PER_W // CHUNK


def _sc_body(user_h, item_h, uf_h, if_h, b_h, bu_h, bi_h, out_h,
             idx_u, idx_i, u_rows, v_rows, bu_v, bi_v, out_v, b_v,
             sem_rows, sem_bias):
    wid = lax.axis_index("s") * NC + lax.axis_index("c")
    base = wid * B_PER_W

    pltpu.sync_copy(user_h.at[pl.ds(wid * N_CHUNKS, N_CHUNKS)], idx_u)
    pltpu.sync_copy(item_h.at[pl.ds(wid * N_CHUNKS, N_CHUNKS)], idx_i)
    pltpu.sync_copy(b_h, b_v)

    copies = []
    for c in range(N_CHUNKS):
        sl = pl.ds(c * CHUNK, CHUNK)
        copies.append(pltpu.async_copy(
            uf_h.at[idx_u.at[c]], u_rows.at[sl], sem_rows))
        copies.append(pltpu.async_copy(
            if_h.at[idx_i.at[c]], v_rows.at[sl], sem_rows))
        copies.append(pltpu.async_copy(
            bu_h.at[idx_u.at[c]], bu_v.at[sl], sem_bias))
        copies.append(pltpu.async_copy(
            bi_h.at[idx_i.at[c]], bi_v.at[sl], sem_bias))
    for cp in copies:
        cp.wait()

    lane = lax.iota(jnp.int32, L)
    b_vec = b_v[...]

    def blk_body(blk, _):
        rbase = blk * L
        rows = rbase + lane
        acc = jnp.zeros((L,), jnp.float32)
        for f in range(F):
            cols = jnp.full((L,), f, jnp.int32)
            uvals = plsc.load_gather(u_rows, [rows, cols])
            vvals = plsc.load_gather(v_rows, [rows, cols])
            acc = acc + uvals * vvals
        score = acc + b_vec + bu_v[pl.ds(rbase, L)] + bi_v[pl.ds(rbase, L)]
        out_v[pl.ds(rbase, L)] = score
        return 0

    lax.fori_loop(0, B_PER_W // L, blk_body, 0)

    pltpu.sync_copy(out_v, out_h.at[pl.ds(base, B_PER_W)])


def kernel(user, item, user_factors, item_factors, b, b_u, b_i):
    bu_flat = b_u.reshape(-1)
    bi_flat = b_i.reshape(-1)
    b16 = jnp.broadcast_to(b, (L,))
    user2d = user.reshape(NW * N_CHUNKS, CHUNK)
    item2d = item.reshape(NW * N_CHUNKS, CHUNK)
    mesh = plsc.VectorSubcoreMesh(core_axis_name="c", subcore_axis_name="s")
    f = pl.kernel(
        _sc_body,
        out_type=jax.ShapeDtypeStruct((BATCH,), jnp.float32),
        mesh=mesh,
        scratch_types=[
            pltpu.VMEM((N_CHUNKS, CHUNK), jnp.int32),
            pltpu.VMEM((N_CHUNKS, CHUNK), jnp.int32),
            pltpu.VMEM((B_PER_W, F), jnp.float32),
            pltpu.VMEM((B_PER_W, F), jnp.float32),
            pltpu.VMEM((B_PER_W,), jnp.float32),
            pltpu.VMEM((B_PER_W,), jnp.float32),
            pltpu.VMEM((B_PER_W,), jnp.float32),
            pltpu.VMEM((L,), jnp.float32),
            pltpu.SemaphoreType.DMA,
            pltpu.SemaphoreType.DMA,
        ],
        compiler_params=pltpu.CompilerParams(
            use_tc_tiling_on_sc=False, needs_layout_passes=False),
    )
    return f(user2d, item2d, user_factors, item_factors, b16, bu_flat, bi_flat)
